# baseline (speedup 1.0000x reference)
import functools

import jax
import jax.numpy as jnp
from jax import lax
from jax.experimental import pallas as pl
from jax.experimental.pallas import tpu as pltpu
from jax.experimental.pallas import tpu_sc as plsc

B, T, H = 1024, 20, 128
SW = 2
GX, GY = 512, 512
NGX, NGY = GX + 3 * SW, GY + 3 * SW
GRID_PAD = 268800
NC, NS = 2, 16
NW = NC * NS
BPW = B // NW
RPW = BPW * 25
NCHUNK = 8
CHW = RPW // NCHUNK
WCH = 8
NROWS = 1 + T * B
_F32 = jnp.float32
_I32 = jnp.int32

_MESH = plsc.VectorSubcoreMesh(
    core_axis_name="c", subcore_axis_name="s", num_cores=NC, num_subcores=NS)


@functools.partial(
    pl.kernel,
    out_type=jax.ShapeDtypeStruct((T, NW, NCHUNK, CHW), _I32),
    mesh=_MESH,
    scratch_types=[
        pltpu.VMEM_SHARED((GRID_PAD,), _I32),
        pltpu.VMEM((NCHUNK, CHW), _I32),
        pltpu.VMEM((NCHUNK, CHW), _I32),
        pltpu.VMEM((WCH, 128), _I32),
        pltpu.VMEM((WCH, 128), _I32),
        pltpu.VMEM((GRID_PAD // NS,), _I32),
        pltpu.SemaphoreType.DMA,
    ],
)
def _resolve(rc_hbm, wc_hbm, rv_hbm, z_hbm, src_hbm,
             grid_sh, cells_v, gath_v, wcell_v, wval_v, zstage_v, sem):
    c = lax.axis_index("c")
    s = lax.axis_index("s")
    w = c * NS + s
    chunk = GRID_PAD // NS
    pltpu.sync_copy(z_hbm.at[pl.ds(s * chunk, chunk)], zstage_v)
    pltpu.sync_copy(zstage_v, grid_sh.at[pl.ds(s * chunk, chunk)])
    plsc.subcore_barrier()

    def body(t, carry):
        pltpu.sync_copy(rc_hbm.at[t, w], cells_v)
        for j in range(NCHUNK):
            pltpu.async_copy(grid_sh.at[cells_v.at[j]], gath_v.at[j], sem).wait()
        pltpu.sync_copy(gath_v, src_hbm.at[t, w])
        plsc.subcore_barrier()

        @pl.when(s == 0)
        def _():
            pltpu.sync_copy(wc_hbm.at[t], wcell_v)
            pltpu.sync_copy(rv_hbm.at[t], wval_v)
            for j in range(WCH):
                pltpu.sync_copy(wval_v.at[j], grid_sh.at[wcell_v.at[j]])
        plsc.subcore_barrier()
        return carry

    lax.fori_loop(0, T, body, 0)


@functools.partial(
    pl.kernel,
    out_type=jax.ShapeDtypeStruct((NW, RPW, H), _F32),
    mesh=_MESH,
    scratch_types=[
        pltpu.VMEM((NCHUNK, CHW), _I32),
        pltpu.VMEM((RPW, H), _F32),
        pltpu.SemaphoreType.DMA,
    ],
)
def _gather(src_hbm, u_hbm, ctx_hbm, idx_v, rows_v, sem):
    c = lax.axis_index("c")
    s = lax.axis_index("s")
    w = c * NS + s
    pltpu.sync_copy(src_hbm.at[w], idx_v)
    cps = [pltpu.async_copy(u_hbm.at[idx_v.at[j]],
                            rows_v.at[pl.ds(j * CHW, CHW)], sem)
           for j in range(NCHUNK)]
    for cp in cps:
        cp.wait()
    pltpu.sync_copy(rows_v, ctx_hbm.at[w])


NBLK = 4
BB = B // NBLK

_HIGH = lax.Precision.HIGHEST


def _step_body(t_ref, u_in, feat, hid, last, sidx,
               wihT, bih, whhT, bhh, linwT, linb, ctx,
               hid_out, last_out, u_out, upd, sem):
    t = t_ref[0]
    i = pl.program_id(0)
    f = feat[...]
    h = hid[...]
    gi = jnp.dot(f, wihT[...], preferred_element_type=_F32, precision=_HIGH) + bih[...]
    gh = jnp.dot(h, whhT[...], preferred_element_type=_F32, precision=_HIGH) + bhh[...]
    i_r, i_i, i_n, i_s = gi[:, :H], gi[:, H:2*H], gi[:, 2*H:3*H], gi[:, 3*H:]
    h_r, h_i, h_n, h_s = gh[:, :H], gh[:, H:2*H], gh[:, 2*H:3*H], gh[:, 3*H:]
    resetg = jax.nn.sigmoid(i_r + h_r)
    updg = jax.nn.sigmoid(i_i + h_i)
    spatg = jax.nn.sigmoid(i_s + h_s)
    newg = jnp.tanh(i_n + resetg * h_n)

    c3 = ctx[...]
    attn = jnp.sum(c3 * newg[:, None, :], axis=2)
    msk = attn == 0.0
    neg = jnp.where(msk, -jnp.inf, attn)
    m = jnp.max(neg, axis=-1, keepdims=True)
    m = jnp.where(jnp.isfinite(m), m, 0.0)
    e = jnp.where(msk, 0.0, jnp.exp(attn - m))
    denom = jnp.sum(e, axis=-1, keepdims=True)
    safe = jnp.where(denom > 0, denom, 1.0)
    p = jnp.where(denom > 0, e / safe, 0.0)
    mix = jnp.sum(p[:, :, None] * c3, axis=1)

    comb = jnp.concatenate([mix, newg], axis=1)
    atten = jnp.tanh(
        jnp.dot(comb, linwT[...], preferred_element_type=_F32, precision=_HIGH)
        + linb[...])
    curr = newg + spatg * atten
    out = curr + updg * (h - curr)
    read = c3[:, 12, :]
    updates = spatg * read + (1.0 - spatg) * out

    hid_out[...] = out
    last_out[...] = jnp.where(sidx[...] == t, out, last[...])
    upd[...] = updates
    pltpu.make_async_copy(
        upd, u_out.at[pl.ds(1 + t * B + i * BB, BB)], sem).start()
    pltpu.make_async_copy(
        upd, u_out.at[pl.ds(1 + t * B + i * BB, BB)], sem).wait()


def _step_call(tarr, u, feat_t, hidden, last, sidx, wihT, bih, whhT, bhh,
               linwT, linb, ctx):
    return pl.pallas_call(
        _step_body,
        grid=(NBLK,),
        in_specs=[
            pl.BlockSpec(memory_space=pltpu.SMEM),
            pl.BlockSpec(memory_space=pltpu.HBM),
            pl.BlockSpec((BB, 2), lambda i: (i, 0)),
            pl.BlockSpec((BB, H), lambda i: (i, 0)),
            pl.BlockSpec((BB, H), lambda i: (i, 0)),
            pl.BlockSpec((BB, 1), lambda i: (i, 0)),
            pl.BlockSpec((2, 4 * H), lambda i: (0, 0)),
            pl.BlockSpec((1, 4 * H), lambda i: (0, 0)),
            pl.BlockSpec((H, 4 * H), lambda i: (0, 0)),
            pl.BlockSpec((1, 4 * H), lambda i: (0, 0)),
            pl.BlockSpec((2 * H, H), lambda i: (0, 0)),
            pl.BlockSpec((1, H), lambda i: (0, 0)),
            pl.BlockSpec((BB, 25, H), lambda i: (i, 0, 0)),
        ],
        out_specs=[
            pl.BlockSpec((BB, H), lambda i: (i, 0)),
            pl.BlockSpec((BB, H), lambda i: (i, 0)),
            pl.BlockSpec(memory_space=pltpu.HBM),
        ],
        out_shape=[
            jax.ShapeDtypeStruct((B, H), _F32),
            jax.ShapeDtypeStruct((B, H), _F32),
            jax.ShapeDtypeStruct((NROWS, H), _F32),
        ],
        input_output_aliases={1: 2},
        scratch_shapes=[pltpu.VMEM((BB, H), _F32), pltpu.SemaphoreType.DMA],
    )(tarr, u, feat_t, hidden, last, sidx, wihT, bih, whhT, bhh, linwT, linb, ctx)


def kernel(feature_tensor, seq_lengths, weight_ih, weight_hh, bias_ih,
           bias_hh, lin_w, lin_b):
    coords = feature_tensor[:, :, 2:4].astype(_I32) + SW
    gx = jnp.clip(coords[:, :, 0], 0, NGX - 1).T
    gy = jnp.clip(coords[:, :, 1], 0, NGY - 1).T
    offs = jnp.arange(-SW, SW + 1, dtype=_I32)
    xi = jnp.clip(gx[:, :, None] + offs, 0, NGX - 1)
    yi = jnp.clip(gy[:, :, None] + offs, 0, NGY - 1)
    read_cells = (xi[:, :, :, None] * NGY + yi[:, :, None, :]).reshape(
        T, NW, NCHUNK, CHW)
    write_cells = (gx * NGY + gy).reshape(T, WCH, 128)
    row_vals = (1 + jnp.arange(T, dtype=_I32)[:, None] * B
                + jnp.arange(B, dtype=_I32)[None, :]).reshape(T, WCH, 128)
    zgrid = jnp.zeros((GRID_PAD,), _I32)

    src = _resolve(read_cells, write_cells, row_vals, zgrid)

    wihT = weight_ih.T
    bih = bias_ih.reshape(1, 4 * H)
    whhT = weight_hh.T
    bhh = bias_hh.reshape(1, 4 * H)
    linwT = lin_w.T
    linb = lin_b.reshape(1, H)
    sidx = (jnp.maximum(seq_lengths, 1) - 1).astype(_I32).reshape(B, 1)

    u = jnp.zeros((NROWS, H), _F32)
    hidden = jnp.zeros((B, H), _F32)
    last = jnp.zeros((B, H), _F32)
    for t in range(T):
        ctx = _gather(src[t], u).reshape(B, 25, H)
        tarr = jnp.full((1,), t, _I32)
        feat_t = feature_tensor[:, t, :2]
        hidden, last, u = _step_call(tarr, u, feat_t, hidden, last, sidx,
                                     wihT, bih, whhT, bhh, linwT, linb, ctx)
    return last


# speedup vs baseline: 13.5101x; 13.5101x over previous
import functools

import jax
import jax.numpy as jnp
from jax import lax
from jax.experimental import pallas as pl
from jax.experimental.pallas import tpu as pltpu
from jax.experimental.pallas import tpu_sc as plsc

B, T, H = 1024, 20, 128
SW = 2
GX, GY = 512, 512
NGX, NGY = GX + 3 * SW, GY + 3 * SW
GRID_PAD = 268800
NC, NS = 2, 16
NW = NC * NS
BPW = B // NW
RPW = BPW * 25
NCHUNK = 8
CHW = RPW // NCHUNK
WCH = 8
NROWS = 1 + T * B
_F32 = jnp.float32
_I32 = jnp.int32

_MESH = plsc.VectorSubcoreMesh(
    core_axis_name="c", subcore_axis_name="s", num_cores=NC, num_subcores=NS)


@functools.partial(
    pl.kernel,
    out_type=jax.ShapeDtypeStruct((T, NW, NCHUNK, CHW), _I32),
    mesh=_MESH,
    scratch_types=[
        pltpu.VMEM_SHARED((GRID_PAD,), _I32),
        pltpu.VMEM((NCHUNK, CHW), _I32),
        pltpu.VMEM((NCHUNK, CHW), _I32),
        pltpu.VMEM((WCH, 128), _I32),
        pltpu.VMEM((WCH, 128), _I32),
        pltpu.VMEM((GRID_PAD // NS,), _I32),
        pltpu.SemaphoreType.DMA,
    ],
)
def _resolve(rc_hbm, wc_hbm, rv_hbm, z_hbm, src_hbm,
             grid_sh, cells_v, gath_v, wcell_v, wval_v, zstage_v, sem):
    c = lax.axis_index("c")
    s = lax.axis_index("s")
    w = c * NS + s
    chunk = GRID_PAD // NS
    pltpu.sync_copy(z_hbm.at[pl.ds(s * chunk, chunk)], zstage_v)
    pltpu.sync_copy(zstage_v, grid_sh.at[pl.ds(s * chunk, chunk)])
    plsc.subcore_barrier()

    def body(t, carry):
        pltpu.sync_copy(rc_hbm.at[t, w], cells_v)
        for j in range(NCHUNK):
            pltpu.async_copy(grid_sh.at[cells_v.at[j]], gath_v.at[j], sem).wait()
        pltpu.sync_copy(gath_v, src_hbm.at[t, w])
        plsc.subcore_barrier()

        @pl.when(s == 0)
        def _():
            pltpu.sync_copy(wc_hbm.at[t], wcell_v)
            pltpu.sync_copy(rv_hbm.at[t], wval_v)
            for j in range(WCH):
                pltpu.sync_copy(wval_v.at[j], grid_sh.at[wcell_v.at[j]])
        plsc.subcore_barrier()
        return carry

    lax.fori_loop(0, T, body, 0)


@functools.partial(
    pl.kernel,
    out_type=jax.ShapeDtypeStruct((NW, RPW, H), _F32),
    mesh=_MESH,
    scratch_types=[
        pltpu.VMEM((RPW,), _I32),
        pltpu.VMEM((RPW, H), _F32),
        pltpu.SemaphoreType.DMA,
    ],
)
def _gather(src_hbm, u_hbm, ctx_hbm, idx_v, rows_v, sem):
    c = lax.axis_index("c")
    s = lax.axis_index("s")
    w = c * NS + s
    pltpu.sync_copy(src_hbm.at[w], idx_v)
    def eff(k):
        raw = idx_v[pl.ds(k * 16, 16)]
        filler = (lax.iota(_I32, 16) + (w * RPW + k * 16)) % NROWS
        return jnp.where(raw == 0, filler, raw)

    cps = [pltpu.async_copy(u_hbm.at[eff(k)],
                            rows_v.at[pl.ds(k * 16, 16)], sem)
           for k in range(RPW // 16)]
    for cp in cps:
        cp.wait()
    pltpu.sync_copy(rows_v, ctx_hbm.at[w])


NBLK = 4
BB = B // NBLK

_HIGH = lax.Precision.HIGHEST


def _step_body(t_ref, u_in, feat, hid, last, sidx,
               wihT, bih, whhT, bhh, linwT, linb, ctx, srcm,
               hid_out, last_out, u_out, upd, sem):
    t = t_ref[0]
    i = pl.program_id(0)
    f = feat[...]
    h = hid[...]
    gi = jnp.dot(f, wihT[...], preferred_element_type=_F32, precision=_HIGH) + bih[...]
    gh = jnp.dot(h, whhT[...], preferred_element_type=_F32, precision=_HIGH) + bhh[...]
    i_r, i_i, i_n, i_s = gi[:, :H], gi[:, H:2*H], gi[:, 2*H:3*H], gi[:, 3*H:]
    h_r, h_i, h_n, h_s = gh[:, :H], gh[:, H:2*H], gh[:, 2*H:3*H], gh[:, 3*H:]
    resetg = jax.nn.sigmoid(i_r + h_r)
    updg = jax.nn.sigmoid(i_i + h_i)
    spatg = jax.nn.sigmoid(i_s + h_s)
    newg = jnp.tanh(i_n + resetg * h_n)

    c3 = jnp.where(srcm[...] == 0, 0.0, ctx[...])
    attn = jnp.sum(c3 * newg[:, None, :], axis=2)
    msk = attn == 0.0
    neg = jnp.where(msk, -jnp.inf, attn)
    m = jnp.max(neg, axis=-1, keepdims=True)
    m = jnp.where(jnp.isfinite(m), m, 0.0)
    e = jnp.where(msk, 0.0, jnp.exp(attn - m))
    denom = jnp.sum(e, axis=-1, keepdims=True)
    safe = jnp.where(denom > 0, denom, 1.0)
    p = jnp.where(denom > 0, e / safe, 0.0)
    mix = jnp.sum(p[:, :, None] * c3, axis=1)

    comb = jnp.concatenate([mix, newg], axis=1)
    atten = jnp.tanh(
        jnp.dot(comb, linwT[...], preferred_element_type=_F32, precision=_HIGH)
        + linb[...])
    curr = newg + spatg * atten
    out = curr + updg * (h - curr)
    read = c3[:, 12, :]
    updates = spatg * read + (1.0 - spatg) * out

    hid_out[...] = out
    last_out[...] = jnp.where(sidx[...] == t, out, last[...])
    upd[...] = updates
    pltpu.make_async_copy(
        upd, u_out.at[pl.ds(1 + t * B + i * BB, BB)], sem).start()
    pltpu.make_async_copy(
        upd, u_out.at[pl.ds(1 + t * B + i * BB, BB)], sem).wait()


def _step_call(tarr, u, feat_t, hidden, last, sidx, wihT, bih, whhT, bhh,
               linwT, linb, ctx, srcm):
    return pl.pallas_call(
        _step_body,
        grid=(NBLK,),
        in_specs=[
            pl.BlockSpec(memory_space=pltpu.SMEM),
            pl.BlockSpec(memory_space=pltpu.HBM),
            pl.BlockSpec((BB, 2), lambda i: (i, 0)),
            pl.BlockSpec((BB, H), lambda i: (i, 0)),
            pl.BlockSpec((BB, H), lambda i: (i, 0)),
            pl.BlockSpec((BB, 1), lambda i: (i, 0)),
            pl.BlockSpec((2, 4 * H), lambda i: (0, 0)),
            pl.BlockSpec((1, 4 * H), lambda i: (0, 0)),
            pl.BlockSpec((H, 4 * H), lambda i: (0, 0)),
            pl.BlockSpec((1, 4 * H), lambda i: (0, 0)),
            pl.BlockSpec((2 * H, H), lambda i: (0, 0)),
            pl.BlockSpec((1, H), lambda i: (0, 0)),
            pl.BlockSpec((BB, 25, H), lambda i: (i, 0, 0)),
            pl.BlockSpec((BB, 25, 1), lambda i: (i, 0, 0)),
        ],
        out_specs=[
            pl.BlockSpec((BB, H), lambda i: (i, 0)),
            pl.BlockSpec((BB, H), lambda i: (i, 0)),
            pl.BlockSpec(memory_space=pltpu.HBM),
        ],
        out_shape=[
            jax.ShapeDtypeStruct((B, H), _F32),
            jax.ShapeDtypeStruct((B, H), _F32),
            jax.ShapeDtypeStruct((NROWS, H), _F32),
        ],
        input_output_aliases={1: 2},
        scratch_shapes=[pltpu.VMEM((BB, H), _F32), pltpu.SemaphoreType.DMA],
    )(tarr, u, feat_t, hidden, last, sidx, wihT, bih, whhT, bhh, linwT, linb,
      ctx, srcm)


def kernel(feature_tensor, seq_lengths, weight_ih, weight_hh, bias_ih,
           bias_hh, lin_w, lin_b):
    coords = feature_tensor[:, :, 2:4].astype(_I32) + SW
    gx = jnp.clip(coords[:, :, 0], 0, NGX - 1).T
    gy = jnp.clip(coords[:, :, 1], 0, NGY - 1).T
    offs = jnp.arange(-SW, SW + 1, dtype=_I32)
    xi = jnp.clip(gx[:, :, None] + offs, 0, NGX - 1)
    yi = jnp.clip(gy[:, :, None] + offs, 0, NGY - 1)
    read_cells = (xi[:, :, :, None] * NGY + yi[:, :, None, :]).reshape(
        T, NW, NCHUNK, CHW)
    write_cells = (gx * NGY + gy).reshape(T, WCH, 128)
    row_vals = (1 + jnp.arange(T, dtype=_I32)[:, None] * B
                + jnp.arange(B, dtype=_I32)[None, :]).reshape(T, WCH, 128)
    zgrid = jnp.zeros((GRID_PAD,), _I32)

    src = _resolve(read_cells, write_cells, row_vals, zgrid)

    wihT = weight_ih.T
    bih = bias_ih.reshape(1, 4 * H)
    whhT = weight_hh.T
    bhh = bias_hh.reshape(1, 4 * H)
    linwT = lin_w.T
    linb = lin_b.reshape(1, H)
    sidx = (jnp.maximum(seq_lengths, 1) - 1).astype(_I32).reshape(B, 1)

    u = jnp.zeros((NROWS, H), _F32)
    hidden = jnp.zeros((B, H), _F32)
    last = jnp.zeros((B, H), _F32)
    for t in range(T):
        ctx = _gather(src[t].reshape(NW, RPW), u).reshape(B, 25, H)
        srcm = src[t].reshape(B, 25, 1)
        tarr = jnp.full((1,), t, _I32)
        feat_t = feature_tensor[:, t, :2]
        hidden, last, u = _step_call(tarr, u, feat_t, hidden, last, sidx,
                                     wihT, bih, whhT, bhh, linwT, linb, ctx,
                                     srcm)
    return last


# speedup vs baseline: 20.8048x; 1.5399x over previous
import functools

import jax
import jax.numpy as jnp
from jax import lax
from jax.experimental import pallas as pl
from jax.experimental.pallas import tpu as pltpu
from jax.experimental.pallas import tpu_sc as plsc

B, T, H = 1024, 20, 128
SW = 2
GX, GY = 512, 512
NGX, NGY = GX + 3 * SW, GY + 3 * SW
GRID_PAD = 268800
NC, NS = 2, 16
NW = NC * NS
BPW = B // NW
RPW = BPW * 25
NCHUNK = 8
CHW = RPW // NCHUNK
WCH = 8
NROWS = 1 + T * B
_F32 = jnp.float32
_I32 = jnp.int32

_MESH = plsc.VectorSubcoreMesh(
    core_axis_name="c", subcore_axis_name="s", num_cores=NC, num_subcores=NS)


@functools.partial(
    pl.kernel,
    out_type=jax.ShapeDtypeStruct((T, NW, NCHUNK, CHW), _I32),
    mesh=_MESH,
    scratch_types=[
        pltpu.VMEM_SHARED((GRID_PAD,), _I32),
        pltpu.VMEM((NCHUNK, CHW), _I32),
        pltpu.VMEM((NCHUNK, CHW), _I32),
        pltpu.VMEM((WCH, 128), _I32),
        pltpu.VMEM((WCH, 128), _I32),
        pltpu.VMEM((GRID_PAD // NS,), _I32),
        pltpu.SemaphoreType.DMA,
    ],
)
def _resolve(rc_hbm, wc_hbm, rv_hbm, z_hbm, src_hbm,
             grid_sh, cells_v, gath_v, wcell_v, wval_v, zstage_v, sem):
    c = lax.axis_index("c")
    s = lax.axis_index("s")
    w = c * NS + s
    chunk = GRID_PAD // NS
    pltpu.sync_copy(z_hbm.at[pl.ds(s * chunk, chunk)], zstage_v)
    pltpu.sync_copy(zstage_v, grid_sh.at[pl.ds(s * chunk, chunk)])
    plsc.subcore_barrier()

    def body(t, carry):
        pltpu.sync_copy(rc_hbm.at[t, w], cells_v)
        for j in range(NCHUNK):
            pltpu.async_copy(grid_sh.at[cells_v.at[j]], gath_v.at[j], sem).wait()
        pltpu.sync_copy(gath_v, src_hbm.at[t, w])
        plsc.subcore_barrier()

        @pl.when(s == 0)
        def _():
            pltpu.sync_copy(wc_hbm.at[t], wcell_v)
            pltpu.sync_copy(rv_hbm.at[t], wval_v)
            for j in range(WCH):
                pltpu.sync_copy(wval_v.at[j], grid_sh.at[wcell_v.at[j]])
        plsc.subcore_barrier()
        return carry

    lax.fori_loop(0, T, body, 0)


KPAD = 32
WAVE = 16


@functools.partial(
    pl.kernel,
    out_type=jax.ShapeDtypeStruct((B, KPAD, H), _F32),
    mesh=_MESH,
    scratch_types=[
        pltpu.VMEM((BPW, 25), _I32),
        pltpu.VMEM((WAVE, KPAD, H), _F32),
        pltpu.SemaphoreType.DMA,
    ],
)
def _gather(src_hbm, u_hbm, ctx_hbm, idx_v, rows_v, sem):
    c = lax.axis_index("c")
    s = lax.axis_index("s")
    w = c * NS + s
    pltpu.sync_copy(src_hbm.at[w], idx_v)
    for wave in range(BPW // WAVE):
        cps = [pltpu.async_copy(u_hbm.at[idx_v.at[wave * WAVE + j]],
                                rows_v.at[j, pl.ds(0, 25)], sem)
               for j in range(WAVE)]
        for cp in cps:
            cp.wait()
        pltpu.sync_copy(rows_v,
                        ctx_hbm.at[pl.ds(w * BPW + wave * WAVE, WAVE)])


NBLK = 4
BB = B // NBLK

_HIGH = lax.Precision.HIGHEST


def _step_body(t_ref, u_in, feat, hid, last, sidx,
               wihT, bih, whhT, bhh, linwT, linb, ctx, src2,
               hid_out, last_out, u_out, upd, ctx_s, sem, sem2):
    t = t_ref[0]
    i = pl.program_id(0)
    cp_in = pltpu.make_async_copy(ctx.at[pl.ds(i * BB, BB)], ctx_s, sem2)
    cp_in.start()
    f = feat[...]
    h = hid[...]
    gi = jnp.dot(f, wihT[...], preferred_element_type=_F32, precision=_HIGH) + bih[...]
    gh = jnp.dot(h, whhT[...], preferred_element_type=_F32, precision=_HIGH) + bhh[...]
    i_r, i_i, i_n, i_s = gi[:, :H], gi[:, H:2*H], gi[:, 2*H:3*H], gi[:, 3*H:]
    h_r, h_i, h_n, h_s = gh[:, :H], gh[:, H:2*H], gh[:, 2*H:3*H], gh[:, 3*H:]
    resetg = jax.nn.sigmoid(i_r + h_r)
    updg = jax.nn.sigmoid(i_i + h_i)
    spatg = jax.nn.sigmoid(i_s + h_s)
    newg = jnp.tanh(i_n + resetg * h_n)

    cp_in.wait()
    c3 = ctx_s[...][:, :25, :]
    attn = jnp.sum(c3 * newg[:, None, :], axis=2)
    attn = jnp.where(src2[...] == 0, 0.0, attn)
    msk = attn == 0.0
    neg = jnp.where(msk, -jnp.inf, attn)
    m = jnp.max(neg, axis=-1, keepdims=True)
    m = jnp.where(jnp.isfinite(m), m, 0.0)
    e = jnp.where(msk, 0.0, jnp.exp(attn - m))
    denom = jnp.sum(e, axis=-1, keepdims=True)
    safe = jnp.where(denom > 0, denom, 1.0)
    p = jnp.where(denom > 0, e / safe, 0.0)
    mix = jnp.sum(p[:, :, None] * c3, axis=1)

    comb = jnp.concatenate([mix, newg], axis=1)
    atten = jnp.tanh(
        jnp.dot(comb, linwT[...], preferred_element_type=_F32, precision=_HIGH)
        + linb[...])
    curr = newg + spatg * atten
    out = curr + updg * (h - curr)
    read = jnp.where(src2[...][:, 12:13] == 0, 0.0, c3[:, 12, :])
    updates = spatg * read + (1.0 - spatg) * out

    hid_out[...] = out
    last_out[...] = jnp.where(sidx[...] == t, out, last[...])
    upd[...] = updates
    pltpu.make_async_copy(
        upd, u_out.at[pl.ds(1 + t * B + i * BB, BB)], sem).start()
    pltpu.make_async_copy(
        upd, u_out.at[pl.ds(1 + t * B + i * BB, BB)], sem).wait()


def _step_call(tarr, u, feat_t, hidden, last, sidx, wihT, bih, whhT, bhh,
               linwT, linb, ctx, srcm):
    return pl.pallas_call(
        _step_body,
        grid=(NBLK,),
        in_specs=[
            pl.BlockSpec(memory_space=pltpu.SMEM),
            pl.BlockSpec(memory_space=pltpu.HBM),
            pl.BlockSpec((BB, 2), lambda i: (i, 0)),
            pl.BlockSpec((BB, H), lambda i: (i, 0)),
            pl.BlockSpec((BB, H), lambda i: (i, 0)),
            pl.BlockSpec((BB, 1), lambda i: (i, 0)),
            pl.BlockSpec((2, 4 * H), lambda i: (0, 0)),
            pl.BlockSpec((1, 4 * H), lambda i: (0, 0)),
            pl.BlockSpec((H, 4 * H), lambda i: (0, 0)),
            pl.BlockSpec((1, 4 * H), lambda i: (0, 0)),
            pl.BlockSpec((2 * H, H), lambda i: (0, 0)),
            pl.BlockSpec((1, H), lambda i: (0, 0)),
            pl.BlockSpec(memory_space=pltpu.HBM),
            pl.BlockSpec((BB, 25), lambda i: (i, 0)),
        ],
        out_specs=[
            pl.BlockSpec((BB, H), lambda i: (i, 0)),
            pl.BlockSpec((BB, H), lambda i: (i, 0)),
            pl.BlockSpec(memory_space=pltpu.HBM),
        ],
        out_shape=[
            jax.ShapeDtypeStruct((B, H), _F32),
            jax.ShapeDtypeStruct((B, H), _F32),
            jax.ShapeDtypeStruct((NROWS, H), _F32),
        ],
        input_output_aliases={1: 2},
        scratch_shapes=[pltpu.VMEM((BB, H), _F32),
                        pltpu.VMEM((BB, KPAD, H), _F32),
                        pltpu.SemaphoreType.DMA, pltpu.SemaphoreType.DMA],
    )(tarr, u, feat_t, hidden, last, sidx, wihT, bih, whhT, bhh, linwT, linb,
      ctx, srcm)


def kernel(feature_tensor, seq_lengths, weight_ih, weight_hh, bias_ih,
           bias_hh, lin_w, lin_b):
    coords = feature_tensor[:, :, 2:4].astype(_I32) + SW
    gx = jnp.clip(coords[:, :, 0], 0, NGX - 1).T
    gy = jnp.clip(coords[:, :, 1], 0, NGY - 1).T
    offs = jnp.arange(-SW, SW + 1, dtype=_I32)
    xi = jnp.clip(gx[:, :, None] + offs, 0, NGX - 1)
    yi = jnp.clip(gy[:, :, None] + offs, 0, NGY - 1)
    read_cells = (xi[:, :, :, None] * NGY + yi[:, :, None, :]).reshape(
        T, NW, NCHUNK, CHW)
    write_cells = (gx * NGY + gy).reshape(T, WCH, 128)
    row_vals = (1 + jnp.arange(T, dtype=_I32)[:, None] * B
                + jnp.arange(B, dtype=_I32)[None, :]).reshape(T, WCH, 128)
    zgrid = jnp.zeros((GRID_PAD,), _I32)

    src = _resolve(read_cells, write_cells, row_vals, zgrid)
    src_all = src.reshape(T, B, 25)
    filler = (jnp.arange(B * 25, dtype=_I32).reshape(B, 25) % (NROWS - 1)) + 1
    src_eff = jnp.where(src_all == 0, filler[None], src_all).reshape(
        T, NW, BPW, 25)

    wihT = weight_ih.T
    bih = bias_ih.reshape(1, 4 * H)
    whhT = weight_hh.T
    bhh = bias_hh.reshape(1, 4 * H)
    linwT = lin_w.T
    linb = lin_b.reshape(1, H)
    sidx = (jnp.maximum(seq_lengths, 1) - 1).astype(_I32).reshape(B, 1)

    u = jnp.zeros((NROWS, H), _F32)
    hidden = jnp.zeros((B, H), _F32)
    last = jnp.zeros((B, H), _F32)
    for t in range(T):
        ctx = _gather(src_eff[t], u)
        tarr = jnp.full((1,), t, _I32)
        feat_t = feature_tensor[:, t, :2]
        hidden, last, u = _step_call(tarr, u, feat_t, hidden, last, sidx,
                                     wihT, bih, whhT, bhh, linwT, linb, ctx,
                                     src_all[t])
    return last


# speedup vs baseline: 21.4781x; 1.0324x over previous
import functools

import jax
import jax.numpy as jnp
from jax import lax
from jax.experimental import pallas as pl
from jax.experimental.pallas import tpu as pltpu
from jax.experimental.pallas import tpu_sc as plsc

B, T, H = 1024, 20, 128
SW = 2
GX, GY = 512, 512
NGX, NGY = GX + 3 * SW, GY + 3 * SW
GRID_PAD = 268800
NC, NS = 2, 16
NW = NC * NS
BPW = B // NW
RPW = BPW * 25
NCHUNK = 8
CHW = RPW // NCHUNK
WCH = 8
NROWS = 1 + T * B
_F32 = jnp.float32
_I32 = jnp.int32

_MESH = plsc.VectorSubcoreMesh(
    core_axis_name="c", subcore_axis_name="s", num_cores=NC, num_subcores=NS)


@functools.partial(
    pl.kernel,
    out_type=jax.ShapeDtypeStruct((T, NW, NCHUNK, CHW), _I32),
    mesh=_MESH,
    scratch_types=[
        pltpu.VMEM_SHARED((GRID_PAD,), _I32),
        pltpu.VMEM((NCHUNK, CHW), _I32),
        pltpu.VMEM((NCHUNK, CHW), _I32),
        pltpu.VMEM((WCH, 128), _I32),
        pltpu.VMEM((WCH, 128), _I32),
        pltpu.VMEM((GRID_PAD // NS,), _I32),
        pltpu.SemaphoreType.DMA,
    ],
)
def _resolve(rc_hbm, wc_hbm, rv_hbm, z_hbm, src_hbm,
             grid_sh, cells_v, gath_v, wcell_v, wval_v, zstage_v, sem):
    c = lax.axis_index("c")
    s = lax.axis_index("s")
    w = c * NS + s
    chunk = GRID_PAD // NS
    pltpu.sync_copy(z_hbm.at[pl.ds(s * chunk, chunk)], zstage_v)
    pltpu.sync_copy(zstage_v, grid_sh.at[pl.ds(s * chunk, chunk)])
    plsc.subcore_barrier()

    def body(t, carry):
        pltpu.sync_copy(rc_hbm.at[t, w], cells_v)
        for j in range(NCHUNK):
            pltpu.async_copy(grid_sh.at[cells_v.at[j]], gath_v.at[j], sem).wait()
        pltpu.sync_copy(gath_v, src_hbm.at[t, w])
        plsc.subcore_barrier()

        @pl.when(s == 0)
        def _():
            pltpu.sync_copy(wc_hbm.at[t], wcell_v)
            pltpu.sync_copy(rv_hbm.at[t], wval_v)
            for j in range(WCH):
                pltpu.sync_copy(wval_v.at[j], grid_sh.at[wcell_v.at[j]])
        plsc.subcore_barrier()
        return carry

    lax.fori_loop(0, T, body, 0)


KPAD = 32
WAVE = 16


@functools.partial(
    pl.kernel,
    out_type=jax.ShapeDtypeStruct((B, KPAD, H), _F32),
    mesh=_MESH,
    scratch_types=[
        pltpu.VMEM((BPW, 25), _I32),
        pltpu.VMEM((WAVE, KPAD, H), _F32),
        pltpu.SemaphoreType.DMA,
    ],
)
def _gather(src_hbm, u_hbm, ctx_hbm, idx_v, rows_v, sem):
    c = lax.axis_index("c")
    s = lax.axis_index("s")
    w = c * NS + s
    pltpu.sync_copy(src_hbm.at[w], idx_v)
    for wave in range(BPW // WAVE):
        cps = [pltpu.async_copy(u_hbm.at[idx_v.at[wave * WAVE + j]],
                                rows_v.at[j, pl.ds(0, 25)], sem)
               for j in range(WAVE)]
        for cp in cps:
            cp.wait()
        pltpu.sync_copy(rows_v,
                        ctx_hbm.at[pl.ds(w * BPW + wave * WAVE, WAVE)])


NBLK = 4
BB = B // NBLK

_HIGH = lax.Precision.DEFAULT


def _step_body(t_ref, u_in, feat, hid, last, sidx,
               wihT, bih, whhT, bhh, linwT, linb, ctx, src2,
               hid_out, last_out, u_out, upd, ctx_s, sem, sem2):
    t = t_ref[0]
    i = pl.program_id(0)
    cp_in = pltpu.make_async_copy(ctx.at[pl.ds(i * BB, BB)], ctx_s, sem2)
    cp_in.start()
    f = feat[...]
    h = hid[...]
    gi = jnp.dot(f, wihT[...], preferred_element_type=_F32, precision=_HIGH) + bih[...]
    gh = jnp.dot(h, whhT[...], preferred_element_type=_F32, precision=_HIGH) + bhh[...]
    i_r, i_i, i_n, i_s = gi[:, :H], gi[:, H:2*H], gi[:, 2*H:3*H], gi[:, 3*H:]
    h_r, h_i, h_n, h_s = gh[:, :H], gh[:, H:2*H], gh[:, 2*H:3*H], gh[:, 3*H:]
    resetg = jax.nn.sigmoid(i_r + h_r)
    updg = jax.nn.sigmoid(i_i + h_i)
    spatg = jax.nn.sigmoid(i_s + h_s)
    newg = jnp.tanh(i_n + resetg * h_n)

    cp_in.wait()
    c3 = ctx_s[...][:, :25, :]
    attn = jnp.sum(c3 * newg[:, None, :], axis=2)
    attn = jnp.where(src2[...] == 0, 0.0, attn)
    msk = attn == 0.0
    neg = jnp.where(msk, -jnp.inf, attn)
    m = jnp.max(neg, axis=-1, keepdims=True)
    m = jnp.where(jnp.isfinite(m), m, 0.0)
    e = jnp.where(msk, 0.0, jnp.exp(attn - m))
    denom = jnp.sum(e, axis=-1, keepdims=True)
    safe = jnp.where(denom > 0, denom, 1.0)
    p = jnp.where(denom > 0, e / safe, 0.0)
    mix = jnp.sum(p[:, :, None] * c3, axis=1)

    comb = jnp.concatenate([mix, newg], axis=1)
    atten = jnp.tanh(
        jnp.dot(comb, linwT[...], preferred_element_type=_F32, precision=_HIGH)
        + linb[...])
    curr = newg + spatg * atten
    out = curr + updg * (h - curr)
    read = jnp.where(src2[...][:, 12:13] == 0, 0.0, c3[:, 12, :])
    updates = spatg * read + (1.0 - spatg) * out

    hid_out[...] = out
    last_out[...] = jnp.where(sidx[...] == t, out, last[...])
    upd[...] = updates
    pltpu.make_async_copy(
        upd, u_out.at[pl.ds(1 + t * B + i * BB, BB)], sem).start()
    pltpu.make_async_copy(
        upd, u_out.at[pl.ds(1 + t * B + i * BB, BB)], sem).wait()


def _step_call(tarr, u, feat_t, hidden, last, sidx, wihT, bih, whhT, bhh,
               linwT, linb, ctx, srcm):
    return pl.pallas_call(
        _step_body,
        grid=(NBLK,),
        in_specs=[
            pl.BlockSpec(memory_space=pltpu.SMEM),
            pl.BlockSpec(memory_space=pltpu.HBM),
            pl.BlockSpec((BB, 2), lambda i: (i, 0)),
            pl.BlockSpec((BB, H), lambda i: (i, 0)),
            pl.BlockSpec((BB, H), lambda i: (i, 0)),
            pl.BlockSpec((BB, 1), lambda i: (i, 0)),
            pl.BlockSpec((2, 4 * H), lambda i: (0, 0)),
            pl.BlockSpec((1, 4 * H), lambda i: (0, 0)),
            pl.BlockSpec((H, 4 * H), lambda i: (0, 0)),
            pl.BlockSpec((1, 4 * H), lambda i: (0, 0)),
            pl.BlockSpec((2 * H, H), lambda i: (0, 0)),
            pl.BlockSpec((1, H), lambda i: (0, 0)),
            pl.BlockSpec(memory_space=pltpu.HBM),
            pl.BlockSpec((BB, 25), lambda i: (i, 0)),
        ],
        out_specs=[
            pl.BlockSpec((BB, H), lambda i: (i, 0)),
            pl.BlockSpec((BB, H), lambda i: (i, 0)),
            pl.BlockSpec(memory_space=pltpu.HBM),
        ],
        out_shape=[
            jax.ShapeDtypeStruct((B, H), _F32),
            jax.ShapeDtypeStruct((B, H), _F32),
            jax.ShapeDtypeStruct((NROWS, H), _F32),
        ],
        input_output_aliases={1: 2},
        scratch_shapes=[pltpu.VMEM((BB, H), _F32),
                        pltpu.VMEM((BB, KPAD, H), _F32),
                        pltpu.SemaphoreType.DMA, pltpu.SemaphoreType.DMA],
    )(tarr, u, feat_t, hidden, last, sidx, wihT, bih, whhT, bhh, linwT, linb,
      ctx, srcm)


def kernel(feature_tensor, seq_lengths, weight_ih, weight_hh, bias_ih,
           bias_hh, lin_w, lin_b):
    coords = feature_tensor[:, :, 2:4].astype(_I32) + SW
    gx = jnp.clip(coords[:, :, 0], 0, NGX - 1).T
    gy = jnp.clip(coords[:, :, 1], 0, NGY - 1).T
    offs = jnp.arange(-SW, SW + 1, dtype=_I32)
    xi = jnp.clip(gx[:, :, None] + offs, 0, NGX - 1)
    yi = jnp.clip(gy[:, :, None] + offs, 0, NGY - 1)
    read_cells = (xi[:, :, :, None] * NGY + yi[:, :, None, :]).reshape(
        T, NW, NCHUNK, CHW)
    write_cells = (gx * NGY + gy).reshape(T, WCH, 128)
    row_vals = (1 + jnp.arange(T, dtype=_I32)[:, None] * B
                + jnp.arange(B, dtype=_I32)[None, :]).reshape(T, WCH, 128)
    zgrid = jnp.zeros((GRID_PAD,), _I32)

    src = _resolve(read_cells, write_cells, row_vals, zgrid)
    src_all = src.reshape(T, B, 25)
    filler = (jnp.arange(B * 25, dtype=_I32).reshape(B, 25) % (NROWS - 1)) + 1
    src_eff = jnp.where(src_all == 0, filler[None], src_all).reshape(
        T, NW, BPW, 25)

    wihT = weight_ih.T
    bih = bias_ih.reshape(1, 4 * H)
    whhT = weight_hh.T
    bhh = bias_hh.reshape(1, 4 * H)
    linwT = lin_w.T
    linb = lin_b.reshape(1, H)
    sidx = (jnp.maximum(seq_lengths, 1) - 1).astype(_I32).reshape(B, 1)

    u = jnp.zeros((NROWS, H), _F32)
    hidden = jnp.zeros((B, H), _F32)
    last = jnp.zeros((B, H), _F32)
    ctx0 = jnp.zeros((B, KPAD, H), _F32)
    for t in range(T):
        ctx = ctx0 if t == 0 else _gather(src_eff[t], u)
        tarr = jnp.full((1,), t, _I32)
        feat_t = feature_tensor[:, t, :2]
        hidden, last, u = _step_call(tarr, u, feat_t, hidden, last, sidx,
                                     wihT, bih, whhT, bhh, linwT, linb, ctx,
                                     src_all[t])
    return last


# speedup vs baseline: 23.5335x; 1.0957x over previous
import functools

import jax
import jax.numpy as jnp
from jax import lax
from jax.experimental import pallas as pl
from jax.experimental.pallas import tpu as pltpu
from jax.experimental.pallas import tpu_sc as plsc

B, T, H = 1024, 20, 128
SW = 2
GX, GY = 512, 512
NGX, NGY = GX + 3 * SW, GY + 3 * SW
GRID_PAD = 268800
NC, NS = 2, 16
NW = NC * NS
BPW = B // NW
RPW = BPW * 25
NCHUNK = 8
CHW = RPW // NCHUNK
WCH = 8
NROWS = 1 + T * B
_F32 = jnp.float32
_I32 = jnp.int32

_MESH = plsc.VectorSubcoreMesh(
    core_axis_name="c", subcore_axis_name="s", num_cores=NC, num_subcores=NS)


@functools.partial(
    pl.kernel,
    out_type=jax.ShapeDtypeStruct((T, NW, NCHUNK, CHW), _I32),
    mesh=_MESH,
    scratch_types=[
        pltpu.VMEM_SHARED((GRID_PAD,), _I32),
        pltpu.VMEM((NCHUNK, CHW), _I32),
        pltpu.VMEM((NCHUNK, CHW), _I32),
        pltpu.VMEM((WCH, 128), _I32),
        pltpu.VMEM((WCH, 128), _I32),
        pltpu.VMEM((GRID_PAD // NS,), _I32),
        pltpu.SemaphoreType.DMA,
    ],
)
def _resolve(rc_hbm, wc_hbm, rv_hbm, z_hbm, src_hbm,
             grid_sh, cells_v, gath_v, wcell_v, wval_v, zstage_v, sem):
    c = lax.axis_index("c")
    s = lax.axis_index("s")
    w = c * NS + s
    chunk = GRID_PAD // NS
    pltpu.sync_copy(z_hbm.at[pl.ds(s * chunk, chunk)], zstage_v)
    pltpu.sync_copy(zstage_v, grid_sh.at[pl.ds(s * chunk, chunk)])
    plsc.subcore_barrier()

    def body(t, carry):
        pltpu.sync_copy(rc_hbm.at[t, w], cells_v)
        for j in range(NCHUNK):
            pltpu.async_copy(grid_sh.at[cells_v.at[j]], gath_v.at[j], sem).wait()
        pltpu.sync_copy(gath_v, src_hbm.at[t, w])
        plsc.subcore_barrier()

        @pl.when(s == 0)
        def _():
            pltpu.sync_copy(wc_hbm.at[t], wcell_v)
            pltpu.sync_copy(rv_hbm.at[t], wval_v)
            for j in range(WCH):
                pltpu.sync_copy(wval_v.at[j], grid_sh.at[wcell_v.at[j]])
        plsc.subcore_barrier()
        return carry

    lax.fori_loop(0, T, body, 0)


KPAD = 32
WAVE = 16


@functools.partial(
    pl.kernel,
    out_type=jax.ShapeDtypeStruct((B, KPAD, H), _F32),
    mesh=_MESH,
    scratch_types=[
        pltpu.VMEM((BPW, 25), _I32),
        pltpu.VMEM((WAVE, KPAD, H), _F32),
        pltpu.SemaphoreType.DMA,
    ],
)
def _gather(src_hbm, u_hbm, ctx_hbm, idx_v, rows_v, sem):
    c = lax.axis_index("c")
    s = lax.axis_index("s")
    w = c * NS + s
    pltpu.sync_copy(src_hbm.at[w], idx_v)
    for wave in range(BPW // WAVE):
        cps = [pltpu.async_copy(u_hbm.at[idx_v.at[wave * WAVE + j]],
                                rows_v.at[j, pl.ds(0, 25)], sem)
               for j in range(WAVE)]
        for cp in cps:
            cp.wait()
        pltpu.sync_copy(rows_v,
                        ctx_hbm.at[pl.ds(w * BPW + wave * WAVE, WAVE)])


NBLK = 2
BB = B // NBLK

_HIGH = lax.Precision.DEFAULT


def _step_body(t_ref, u_in, feat, hid, last, sidx,
               wihT, bih, whhT, bhh, linwT, linb, ctx, src2,
               hid_out, last_out, u_out, upd, ctx_s, sem, sem2):
    t = t_ref[0]
    i = pl.program_id(0)
    cp_in = pltpu.make_async_copy(ctx.at[pl.ds(i * BB, BB)], ctx_s, sem2)
    cp_in.start()
    f = feat[...]
    h = hid[...]
    gi = jnp.dot(f, wihT[...], preferred_element_type=_F32, precision=_HIGH) + bih[...]
    gh = jnp.dot(h, whhT[...], preferred_element_type=_F32, precision=_HIGH) + bhh[...]
    i_r, i_i, i_n, i_s = gi[:, :H], gi[:, H:2*H], gi[:, 2*H:3*H], gi[:, 3*H:]
    h_r, h_i, h_n, h_s = gh[:, :H], gh[:, H:2*H], gh[:, 2*H:3*H], gh[:, 3*H:]
    resetg = jax.nn.sigmoid(i_r + h_r)
    updg = jax.nn.sigmoid(i_i + h_i)
    spatg = jax.nn.sigmoid(i_s + h_s)
    newg = jnp.tanh(i_n + resetg * h_n)

    cp_in.wait()
    c3 = ctx_s[...][:, :25, :]
    attn = jnp.sum(c3 * newg[:, None, :], axis=2)
    attn = jnp.where(src2[...] == 0, 0.0, attn)
    msk = attn == 0.0
    neg = jnp.where(msk, -jnp.inf, attn)
    m = jnp.max(neg, axis=-1, keepdims=True)
    m = jnp.where(jnp.isfinite(m), m, 0.0)
    e = jnp.where(msk, 0.0, jnp.exp(attn - m))
    denom = jnp.sum(e, axis=-1, keepdims=True)
    safe = jnp.where(denom > 0, denom, 1.0)
    p = jnp.where(denom > 0, e / safe, 0.0)
    mix = jnp.sum(p[:, :, None] * c3, axis=1)

    comb = jnp.concatenate([mix, newg], axis=1)
    atten = jnp.tanh(
        jnp.dot(comb, linwT[...], preferred_element_type=_F32, precision=_HIGH)
        + linb[...])
    curr = newg + spatg * atten
    out = curr + updg * (h - curr)
    read = jnp.where(src2[...][:, 12:13] == 0, 0.0, c3[:, 12, :])
    updates = spatg * read + (1.0 - spatg) * out

    hid_out[...] = out
    last_out[...] = jnp.where(sidx[...] == t, out, last[...])
    upd[...] = updates
    pltpu.make_async_copy(
        upd, u_out.at[pl.ds(1 + t * B + i * BB, BB)], sem).start()
    pltpu.make_async_copy(
        upd, u_out.at[pl.ds(1 + t * B + i * BB, BB)], sem).wait()


def _step_call(tarr, u, feat_t, hidden, last, sidx, wihT, bih, whhT, bhh,
               linwT, linb, ctx, srcm):
    return pl.pallas_call(
        _step_body,
        grid=(NBLK,),
        in_specs=[
            pl.BlockSpec(memory_space=pltpu.SMEM),
            pl.BlockSpec(memory_space=pltpu.HBM),
            pl.BlockSpec((BB, 2), lambda i: (i, 0)),
            pl.BlockSpec((BB, H), lambda i: (i, 0)),
            pl.BlockSpec((BB, H), lambda i: (i, 0)),
            pl.BlockSpec((BB, 1), lambda i: (i, 0)),
            pl.BlockSpec((2, 4 * H), lambda i: (0, 0)),
            pl.BlockSpec((1, 4 * H), lambda i: (0, 0)),
            pl.BlockSpec((H, 4 * H), lambda i: (0, 0)),
            pl.BlockSpec((1, 4 * H), lambda i: (0, 0)),
            pl.BlockSpec((2 * H, H), lambda i: (0, 0)),
            pl.BlockSpec((1, H), lambda i: (0, 0)),
            pl.BlockSpec(memory_space=pltpu.HBM),
            pl.BlockSpec((BB, 25), lambda i: (i, 0)),
        ],
        out_specs=[
            pl.BlockSpec((BB, H), lambda i: (i, 0)),
            pl.BlockSpec((BB, H), lambda i: (i, 0)),
            pl.BlockSpec(memory_space=pltpu.HBM),
        ],
        out_shape=[
            jax.ShapeDtypeStruct((B, H), _F32),
            jax.ShapeDtypeStruct((B, H), _F32),
            jax.ShapeDtypeStruct((NROWS, H), _F32),
        ],
        input_output_aliases={1: 2},
        scratch_shapes=[pltpu.VMEM((BB, H), _F32),
                        pltpu.VMEM((BB, KPAD, H), _F32),
                        pltpu.SemaphoreType.DMA, pltpu.SemaphoreType.DMA],
    )(tarr, u, feat_t, hidden, last, sidx, wihT, bih, whhT, bhh, linwT, linb,
      ctx, srcm)


def kernel(feature_tensor, seq_lengths, weight_ih, weight_hh, bias_ih,
           bias_hh, lin_w, lin_b):
    coords = feature_tensor[:, :, 2:4].astype(_I32) + SW
    gx = jnp.clip(coords[:, :, 0], 0, NGX - 1).T
    gy = jnp.clip(coords[:, :, 1], 0, NGY - 1).T
    offs = jnp.arange(-SW, SW + 1, dtype=_I32)
    xi = jnp.clip(gx[:, :, None] + offs, 0, NGX - 1)
    yi = jnp.clip(gy[:, :, None] + offs, 0, NGY - 1)
    read_cells = (xi[:, :, :, None] * NGY + yi[:, :, None, :]).reshape(
        T, NW, NCHUNK, CHW)
    write_cells = (gx * NGY + gy).reshape(T, WCH, 128)
    row_vals = (1 + jnp.arange(T, dtype=_I32)[:, None] * B
                + jnp.arange(B, dtype=_I32)[None, :]).reshape(T, WCH, 128)
    zgrid = jnp.zeros((GRID_PAD,), _I32)

    src = _resolve(read_cells, write_cells, row_vals, zgrid)
    src_all = src.reshape(T, B, 25)
    filler = (jnp.arange(B * 25, dtype=_I32).reshape(B, 25) % (NROWS - 1)) + 1
    src_eff = jnp.where(src_all == 0, filler[None], src_all).reshape(
        T, NW, BPW, 25)

    wihT = weight_ih.T
    bih = bias_ih.reshape(1, 4 * H)
    whhT = weight_hh.T
    bhh = bias_hh.reshape(1, 4 * H)
    linwT = lin_w.T
    linb = lin_b.reshape(1, H)
    sidx = (jnp.maximum(seq_lengths, 1) - 1).astype(_I32).reshape(B, 1)

    u = jnp.zeros((NROWS, H), _F32)
    hidden = jnp.zeros((B, H), _F32)
    last = jnp.zeros((B, H), _F32)
    ctx0 = jnp.zeros((B, KPAD, H), _F32)
    for t in range(T):
        ctx = ctx0 if t == 0 else _gather(src_eff[t], u)
        tarr = jnp.full((1,), t, _I32)
        feat_t = feature_tensor[:, t, :2]
        hidden, last, u = _step_call(tarr, u, feat_t, hidden, last, sidx,
                                     wihT, bih, whhT, bhh, linwT, linb, ctx,
                                     src_all[t])
    return last


# speedup vs baseline: 24.3731x; 1.0357x over previous
import functools

import jax
import jax.numpy as jnp
from jax import lax
from jax.experimental import pallas as pl
from jax.experimental.pallas import tpu as pltpu
from jax.experimental.pallas import tpu_sc as plsc

B, T, H = 1024, 20, 128
SW = 2
GX, GY = 512, 512
NGX, NGY = GX + 3 * SW, GY + 3 * SW
GRID_PAD = 268800
NC, NS = 2, 16
NW = NC * NS
BPW = B // NW
RPW = BPW * 25
NCHUNK = 8
CHW = RPW // NCHUNK
WCH = 8
NROWS = 1 + T * B
_F32 = jnp.float32
_I32 = jnp.int32

_MESH = plsc.VectorSubcoreMesh(
    core_axis_name="c", subcore_axis_name="s", num_cores=NC, num_subcores=NS)


@functools.partial(
    pl.kernel,
    out_type=jax.ShapeDtypeStruct((T, NW, NCHUNK, CHW), _I32),
    mesh=_MESH,
    scratch_types=[
        pltpu.VMEM_SHARED((GRID_PAD,), _I32),
        pltpu.VMEM((NCHUNK, CHW), _I32),
        pltpu.VMEM((NCHUNK, CHW), _I32),
        pltpu.VMEM((WCH, 128), _I32),
        pltpu.VMEM((WCH, 128), _I32),
        pltpu.VMEM((GRID_PAD // NS,), _I32),
        pltpu.SemaphoreType.DMA,
    ],
)
def _resolve(rc_hbm, wc_hbm, rv_hbm, z_hbm, src_hbm,
             grid_sh, cells_v, gath_v, wcell_v, wval_v, zstage_v, sem):
    c = lax.axis_index("c")
    s = lax.axis_index("s")
    w = c * NS + s
    chunk = GRID_PAD // NS
    pltpu.sync_copy(z_hbm.at[pl.ds(s * chunk, chunk)], zstage_v)
    pltpu.sync_copy(zstage_v, grid_sh.at[pl.ds(s * chunk, chunk)])
    plsc.subcore_barrier()

    def body(t, carry):
        pltpu.sync_copy(rc_hbm.at[t, w], cells_v)
        for j in range(NCHUNK):
            pltpu.async_copy(grid_sh.at[cells_v.at[j]], gath_v.at[j], sem).wait()
        pltpu.sync_copy(gath_v, src_hbm.at[t, w])
        plsc.subcore_barrier()

        @pl.when(s == 0)
        def _():
            pltpu.sync_copy(wc_hbm.at[t], wcell_v)
            pltpu.sync_copy(rv_hbm.at[t], wval_v)
            for j in range(WCH):
                pltpu.sync_copy(wval_v.at[j], grid_sh.at[wcell_v.at[j]])
        plsc.subcore_barrier()
        return carry

    lax.fori_loop(0, T, body, 0)


KPAD = 32
WAVE = 16


@functools.partial(
    pl.kernel,
    out_type=jax.ShapeDtypeStruct((B, KPAD, H), _F32),
    mesh=_MESH,
    scratch_types=[
        pltpu.VMEM((BPW, 25), _I32),
        pltpu.VMEM((WAVE, KPAD, H), _F32),
        pltpu.SemaphoreType.DMA,
    ],
)
def _gather(src_hbm, u_hbm, ctx_hbm, idx_v, rows_v, sem):
    c = lax.axis_index("c")
    s = lax.axis_index("s")
    w = c * NS + s
    pltpu.sync_copy(src_hbm.at[w], idx_v)
    for wave in range(BPW // WAVE):
        cps = [pltpu.async_copy(u_hbm.at[idx_v.at[wave * WAVE + j]],
                                rows_v.at[j, pl.ds(0, 25)], sem)
               for j in range(WAVE)]
        for cp in cps:
            cp.wait()
        pltpu.sync_copy(rows_v,
                        ctx_hbm.at[pl.ds(w * BPW + wave * WAVE, WAVE)])


NBLK = 1
BB = B // NBLK

_HIGH = lax.Precision.DEFAULT


def _step_body(t_ref, u_in, feat, hid, last, sidx,
               wihT, bih, whhT, bhh, linwT, linb, ctx, src2,
               hid_out, last_out, u_out, upd, ctx_s, sem, sem2):
    t = t_ref[0]
    i = pl.program_id(0)
    cp_in = pltpu.make_async_copy(ctx.at[pl.ds(i * BB, BB)], ctx_s, sem2)
    cp_in.start()
    f = feat[...]
    h = hid[...]
    gi = jnp.dot(f, wihT[...], preferred_element_type=_F32, precision=_HIGH) + bih[...]
    gh = jnp.dot(h, whhT[...], preferred_element_type=_F32, precision=_HIGH) + bhh[...]
    i_r, i_i, i_n, i_s = gi[:, :H], gi[:, H:2*H], gi[:, 2*H:3*H], gi[:, 3*H:]
    h_r, h_i, h_n, h_s = gh[:, :H], gh[:, H:2*H], gh[:, 2*H:3*H], gh[:, 3*H:]
    resetg = jax.nn.sigmoid(i_r + h_r)
    updg = jax.nn.sigmoid(i_i + h_i)
    spatg = jax.nn.sigmoid(i_s + h_s)
    newg = jnp.tanh(i_n + resetg * h_n)

    cp_in.wait()
    c3 = ctx_s[...][:, :25, :]
    attn = jnp.sum(c3 * newg[:, None, :], axis=2)
    attn = jnp.where(src2[...] == 0, 0.0, attn)
    msk = attn == 0.0
    neg = jnp.where(msk, -jnp.inf, attn)
    m = jnp.max(neg, axis=-1, keepdims=True)
    m = jnp.where(jnp.isfinite(m), m, 0.0)
    e = jnp.where(msk, 0.0, jnp.exp(attn - m))
    denom = jnp.sum(e, axis=-1, keepdims=True)
    safe = jnp.where(denom > 0, denom, 1.0)
    p = jnp.where(denom > 0, e / safe, 0.0)
    mix = jnp.sum(p[:, :, None] * c3, axis=1)

    comb = jnp.concatenate([mix, newg], axis=1)
    atten = jnp.tanh(
        jnp.dot(comb, linwT[...], preferred_element_type=_F32, precision=_HIGH)
        + linb[...])
    curr = newg + spatg * atten
    out = curr + updg * (h - curr)
    read = jnp.where(src2[...][:, 12:13] == 0, 0.0, c3[:, 12, :])
    updates = spatg * read + (1.0 - spatg) * out

    hid_out[...] = out
    last_out[...] = jnp.where(sidx[...] == t, out, last[...])
    upd[...] = updates
    pltpu.make_async_copy(
        upd, u_out.at[pl.ds(1 + t * B + i * BB, BB)], sem).start()
    pltpu.make_async_copy(
        upd, u_out.at[pl.ds(1 + t * B + i * BB, BB)], sem).wait()


def _step_call(tarr, u, feat_t, hidden, last, sidx, wihT, bih, whhT, bhh,
               linwT, linb, ctx, srcm):
    return pl.pallas_call(
        _step_body,
        grid=(NBLK,),
        in_specs=[
            pl.BlockSpec(memory_space=pltpu.SMEM),
            pl.BlockSpec(memory_space=pltpu.HBM),
            pl.BlockSpec((BB, 2), lambda i: (i, 0)),
            pl.BlockSpec((BB, H), lambda i: (i, 0)),
            pl.BlockSpec((BB, H), lambda i: (i, 0)),
            pl.BlockSpec((BB, 1), lambda i: (i, 0)),
            pl.BlockSpec((2, 4 * H), lambda i: (0, 0)),
            pl.BlockSpec((1, 4 * H), lambda i: (0, 0)),
            pl.BlockSpec((H, 4 * H), lambda i: (0, 0)),
            pl.BlockSpec((1, 4 * H), lambda i: (0, 0)),
            pl.BlockSpec((2 * H, H), lambda i: (0, 0)),
            pl.BlockSpec((1, H), lambda i: (0, 0)),
            pl.BlockSpec(memory_space=pltpu.HBM),
            pl.BlockSpec((BB, 25), lambda i: (i, 0)),
        ],
        out_specs=[
            pl.BlockSpec((BB, H), lambda i: (i, 0)),
            pl.BlockSpec((BB, H), lambda i: (i, 0)),
            pl.BlockSpec(memory_space=pltpu.HBM),
        ],
        out_shape=[
            jax.ShapeDtypeStruct((B, H), _F32),
            jax.ShapeDtypeStruct((B, H), _F32),
            jax.ShapeDtypeStruct((NROWS, H), _F32),
        ],
        input_output_aliases={1: 2},
        scratch_shapes=[pltpu.VMEM((BB, H), _F32),
                        pltpu.VMEM((BB, KPAD, H), _F32),
                        pltpu.SemaphoreType.DMA, pltpu.SemaphoreType.DMA],
    )(tarr, u, feat_t, hidden, last, sidx, wihT, bih, whhT, bhh, linwT, linb,
      ctx, srcm)


def kernel(feature_tensor, seq_lengths, weight_ih, weight_hh, bias_ih,
           bias_hh, lin_w, lin_b):
    coords = feature_tensor[:, :, 2:4].astype(_I32) + SW
    gx = jnp.clip(coords[:, :, 0], 0, NGX - 1).T
    gy = jnp.clip(coords[:, :, 1], 0, NGY - 1).T
    offs = jnp.arange(-SW, SW + 1, dtype=_I32)
    xi = jnp.clip(gx[:, :, None] + offs, 0, NGX - 1)
    yi = jnp.clip(gy[:, :, None] + offs, 0, NGY - 1)
    read_cells = (xi[:, :, :, None] * NGY + yi[:, :, None, :]).reshape(
        T, NW, NCHUNK, CHW)
    write_cells = (gx * NGY + gy).reshape(T, WCH, 128)
    row_vals = (1 + jnp.arange(T, dtype=_I32)[:, None] * B
                + jnp.arange(B, dtype=_I32)[None, :]).reshape(T, WCH, 128)
    zgrid = jnp.zeros((GRID_PAD,), _I32)

    src = _resolve(read_cells, write_cells, row_vals, zgrid)
    src_all = src.reshape(T, B, 25)
    filler = (jnp.arange(B * 25, dtype=_I32).reshape(B, 25) % (NROWS - 1)) + 1
    src_eff = jnp.where(src_all == 0, filler[None], src_all).reshape(
        T, NW, BPW, 25)

    wihT = weight_ih.T
    bih = bias_ih.reshape(1, 4 * H)
    whhT = weight_hh.T
    bhh = bias_hh.reshape(1, 4 * H)
    linwT = lin_w.T
    linb = lin_b.reshape(1, H)
    sidx = (jnp.maximum(seq_lengths, 1) - 1).astype(_I32).reshape(B, 1)

    u = jnp.zeros((NROWS, H), _F32)
    hidden = jnp.zeros((B, H), _F32)
    last = jnp.zeros((B, H), _F32)
    ctx0 = jnp.zeros((B, KPAD, H), _F32)
    for t in range(T):
        ctx = ctx0 if t == 0 else _gather(src_eff[t], u)
        tarr = jnp.full((1,), t, _I32)
        feat_t = feature_tensor[:, t, :2]
        hidden, last, u = _step_call(tarr, u, feat_t, hidden, last, sidx,
                                     wihT, bih, whhT, bhh, linwT, linb, ctx,
                                     src_all[t])
    return last
